# Initial kernel scaffold; baseline (speedup 1.0000x reference)
import jax
import jax.numpy as jnp
from jax.experimental import pallas as pl


def kernel(x, labels_pointer, emb_table):
    raise NotImplementedError("write your pallas kernel here")


# speedup vs baseline: 6.2330x; 6.2330x over previous
import jax
import jax.numpy as jnp
from jax.experimental import pallas as pl
from jax.experimental.pallas import tpu as pltpu

B, L, D = 1024, 200, 128
BB = 8


def _body(labels_ref, x_ref, emb_ref, out_ref):
    i = pl.program_id(0)
    out_ref[:, :, :D] = x_ref[...]
    for j in range(BB):
        label = labels_ref[i * BB + j]
        row = emb_ref[label, :]
        out_ref[j, :, D:] = jnp.broadcast_to(row[None, :], (L, D))


def kernel(x, labels_pointer, emb_table):
    grid_spec = pltpu.PrefetchScalarGridSpec(
        num_scalar_prefetch=1,
        grid=(B // BB,),
        in_specs=[
            pl.BlockSpec((BB, L, D), lambda i, *_: (i, 0, 0)),
            pl.BlockSpec((emb_table.shape[0], D), lambda i, *_: (0, 0)),
        ],
        out_specs=pl.BlockSpec((BB, L, 2 * D), lambda i, *_: (i, 0, 0)),
    )
    return pl.pallas_call(
        _body,
        grid_spec=grid_spec,
        out_shape=jax.ShapeDtypeStruct((B, L, 2 * D), x.dtype),
    )(labels_pointer, x, emb_table)


# speedup vs baseline: 8.1643x; 1.3099x over previous
import jax
import jax.numpy as jnp
from jax.experimental import pallas as pl
from jax.experimental.pallas import tpu as pltpu

B, L, D = 1024, 200, 128
BB = 16


def _body(labels_ref, x_ref, emb_ref, out_ref):
    i = pl.program_id(0)
    out_ref[:, :, :D] = x_ref[...]
    for j in range(BB):
        label = labels_ref[i * BB + j]
        row = emb_ref[label, :]
        out_ref[j, :, D:] = jnp.broadcast_to(row[None, :], (L, D))


def kernel(x, labels_pointer, emb_table):
    grid_spec = pltpu.PrefetchScalarGridSpec(
        num_scalar_prefetch=1,
        grid=(B // BB,),
        in_specs=[
            pl.BlockSpec((BB, L, D), lambda i, *_: (i, 0, 0)),
            pl.BlockSpec((emb_table.shape[0], D), lambda i, *_: (0, 0)),
        ],
        out_specs=pl.BlockSpec((BB, L, 2 * D), lambda i, *_: (i, 0, 0)),
    )
    return pl.pallas_call(
        _body,
        grid_spec=grid_spec,
        out_shape=jax.ShapeDtypeStruct((B, L, 2 * D), x.dtype),
        compiler_params=pltpu.CompilerParams(
            dimension_semantics=("parallel",),
        ),
    )(labels_pointer, x, emb_table)


# speedup vs baseline: 8.6169x; 1.0554x over previous
import jax
import jax.numpy as jnp
from jax.experimental import pallas as pl
from jax.experimental.pallas import tpu as pltpu

B, L, D = 1024, 200, 128
BB = 32


def _body(labels_ref, x_ref, emb_ref, out_ref):
    i = pl.program_id(0)
    out_ref[:, :, :D] = x_ref[...]
    for j in range(BB):
        label = labels_ref[i * BB + j]
        row = emb_ref[label, :]
        out_ref[j, :, D:] = jnp.broadcast_to(row[None, :], (L, D))


def kernel(x, labels_pointer, emb_table):
    grid_spec = pltpu.PrefetchScalarGridSpec(
        num_scalar_prefetch=1,
        grid=(B // BB,),
        in_specs=[
            pl.BlockSpec((BB, L, D), lambda i, *_: (i, 0, 0)),
            pl.BlockSpec((emb_table.shape[0], D), lambda i, *_: (0, 0)),
        ],
        out_specs=pl.BlockSpec((BB, L, 2 * D), lambda i, *_: (i, 0, 0)),
    )
    return pl.pallas_call(
        _body,
        grid_spec=grid_spec,
        out_shape=jax.ShapeDtypeStruct((B, L, 2 * D), x.dtype),
        compiler_params=pltpu.CompilerParams(
            dimension_semantics=("parallel",),
        ),
    )(labels_pointer, x, emb_table)


# speedup vs baseline: 8.8975x; 1.0326x over previous
import jax
import jax.numpy as jnp
from jax.experimental import pallas as pl
from jax.experimental.pallas import tpu as pltpu

B, L, D = 1024, 200, 128
BB = 64


def _body(labels_ref, x_ref, emb_ref, out_ref):
    i = pl.program_id(0)
    out_ref[:, :, :D] = x_ref[...]
    for j in range(BB):
        label = labels_ref[i * BB + j]
        row = emb_ref[label, :]
        out_ref[j, :, D:] = jnp.broadcast_to(row[None, :], (L, D))


def kernel(x, labels_pointer, emb_table):
    grid_spec = pltpu.PrefetchScalarGridSpec(
        num_scalar_prefetch=1,
        grid=(B // BB,),
        in_specs=[
            pl.BlockSpec((BB, L, D), lambda i, *_: (i, 0, 0)),
            pl.BlockSpec((emb_table.shape[0], D), lambda i, *_: (0, 0)),
        ],
        out_specs=pl.BlockSpec((BB, L, 2 * D), lambda i, *_: (i, 0, 0)),
    )
    return pl.pallas_call(
        _body,
        grid_spec=grid_spec,
        out_shape=jax.ShapeDtypeStruct((B, L, 2 * D), x.dtype),
        compiler_params=pltpu.CompilerParams(
            dimension_semantics=("parallel",),
        ),
    )(labels_pointer, x, emb_table)
